# baseline (speedup 1.0000x reference)
import functools

import numpy as np
import jax
import jax.numpy as jnp
from jax import lax
from jax.experimental import pallas as pl
from jax.experimental.pallas import tpu as pltpu
from jax.experimental.pallas import tpu_sc as plsc

N_BATCH = 16
N_CHAN = 512
N_COL = 4096

SC_BATCH = 10
TC_BATCH = N_BATCH - SC_BATCH

NC = 2
NS = 16
NW = NC * NS
K = 8
NBUF = 3

_mesh = plsc.VectorSubcoreMesh(core_axis_name="c", subcore_axis_name="s")


def _make_sc_reverse(n_batch):
    rows = n_batch * N_CHAN
    rows_per_tile = rows // NW
    chunks = rows_per_tile // K
    assert rows_per_tile * NW == rows and chunks * K == rows_per_tile

    @functools.partial(
        pl.kernel,
        mesh=_mesh,
        out_type=jax.ShapeDtypeStruct((rows, N_COL), jnp.float32),
        scratch_types=[
            pltpu.VMEM((rows_per_tile,), jnp.int32),
            pltpu.VMEM((NBUF, K, N_COL), jnp.float32),
            pltpu.SemaphoreType.DMA((NBUF,)),
            pltpu.SemaphoreType.DMA((NBUF,)),
        ],
    )
    def sc_reverse(x_hbm, idx_hbm, out_hbm, idx_v, bufs, gsem, ssem):
        wid = lax.axis_index("s") * NC + lax.axis_index("c")
        base = wid * rows_per_tile
        pltpu.sync_copy(idx_hbm.at[pl.ds(base, rows_per_tile)], idx_v)

        gathers = [None] * chunks
        stores = [None] * chunks

        def fire_gather(c):
            b = c % NBUF
            if stores[c - NBUF] is not None:
                stores[c - NBUF].wait()
            gathers[c] = pltpu.async_copy(
                x_hbm.at[idx_v.at[pl.ds(c * K, K)]], bufs.at[b], gsem.at[b])

        fire_gather(0)
        fire_gather(1)
        for c in range(chunks):
            if c + 2 < chunks:
                fire_gather(c + 2)
            b = c % NBUF
            gathers[c].wait()
            stores[c] = pltpu.async_copy(
                bufs.at[b], out_hbm.at[pl.ds(base + c * K, K)], ssem.at[b])
        for c in range(chunks - NBUF, chunks):
            stores[c].wait()

    return sc_reverse


_sc_reverse = _make_sc_reverse(SC_BATCH)

_IDX_NP = np.bitwise_xor(
    np.arange(SC_BATCH * N_CHAN, dtype=np.int32), N_CHAN - 1)


CB = 64
NCB = N_CHAN // CB

_P_FLIP = np.zeros((CB, CB), dtype=np.float32)
_P_FLIP[np.arange(CB), CB - 1 - np.arange(CB)] = 1.0


def _tc_body(p_ref, in_ref, out_ref):
    out_ref[0] = jax.lax.dot(
        p_ref[...], in_ref[0], precision=jax.lax.Precision.HIGHEST)


def _tc_reverse(x):
    return pl.pallas_call(
        _tc_body,
        grid=(TC_BATCH, NCB),
        in_specs=[
            pl.BlockSpec((CB, CB), lambda b, c: (0, 0)),
            pl.BlockSpec(
                (1, CB, N_COL),
                lambda b, c: (SC_BATCH + b, NCB - 1 - c, 0)),
        ],
        out_specs=pl.BlockSpec((1, CB, N_COL), lambda b, c: (b, c, 0)),
        out_shape=jax.ShapeDtypeStruct((TC_BATCH, N_CHAN, N_COL),
                                       jnp.float32),
    )(jnp.asarray(_P_FLIP), x)


def kernel(x, cond):
    del cond
    xf = x.reshape(N_BATCH * N_CHAN, N_COL)
    z_sc = _sc_reverse(xf, jnp.asarray(_IDX_NP))
    z_tc = _tc_reverse(x)
    z = jnp.concatenate(
        [z_sc.reshape(SC_BATCH, N_CHAN, N_COL), z_tc], axis=0)
    log_det_J = jnp.zeros((1,), dtype=jnp.float32)
    return (z, log_det_J)
